# baseline (speedup 1.0000x reference)
import functools

import jax
import jax.numpy as jnp
from jax import lax
from jax.experimental import pallas as pl
from jax.experimental.pallas import tpu as pltpu
from jax.experimental.pallas import tpu_sc as plsc

N = 100000
E = 1600000
NC = 2
NS = 16
NW = NC * NS
D = 16
CH = 800
NCHUNK = E // NS // CH
EPT = E // NS
WCH = 250
NPT = N // NS
DCH = 1000
DNCHUNK = EPT // DCH

_f32 = jnp.float32


def _sc_mesh():
    return plsc.VectorSubcoreMesh(
        core_axis_name="c", subcore_axis_name="s", num_cores=NC, num_subcores=NS
    )


_SC_PARAMS = pltpu.CompilerParams(
    use_tc_tiling_on_sc=False, needs_layout_passes=False)


def _rsqrt_nr(v):
    i = plsc.bitcast(v, jnp.int32)
    y = plsc.bitcast(jnp.int32(0x5F3759DF) - (i >> 1), _f32)
    for _ in range(4):
        y = y * (1.5 - 0.5 * v * y * y)
    return y


def _deg_kernel(dst, ones16, d1e, d2e, shared, didx0, didx1,
                sbufd, d1b, d2b, obuf, is0, is1):
    c = lax.axis_index("c")
    s = lax.axis_index("s")
    wid = c * NS + s
    pltpu.sync_copy(ones16, obuf)

    def zb(j, _):
        d1b[j] = jnp.zeros((16,), _f32)
        return ()

    lax.fori_loop(0, WCH, zb, ())
    for t in range(NPT // WCH):
        pltpu.sync_copy(d1b, shared.at[pl.ds(s * NPT + t * WCH, WCH), :])
    plsc.subcore_barrier()

    dbs = (didx0, didx1)
    iss = (is0, is1)

    def idx_start(i, p):
        base = s * EPT + i * DCH
        return pltpu.async_copy(dst.at[pl.ds(base, DCH)], dbs[p], iss[p])

    def idx_wait(i, p):
        base = s * EPT + i * DCH
        pltpu.make_async_copy(dst.at[pl.ds(base, DCH)], dbs[p], iss[p]).wait()

    def scat(p):
        pltpu.sync_copy(obuf, shared.at[dbs[p]], add=True)

    idx_start(0, 0)

    def pair(k, _):
        i0 = 2 * k
        idx_wait(i0, 0)
        idx_start(i0 + 1, 1)
        scat(0)
        idx_wait(i0 + 1, 1)
        idx_start(i0 + 2, 0)
        scat(1)
        return ()

    assert DNCHUNK % 2 == 0
    lax.fori_loop(0, DNCHUNK // 2 - 1, pair, ())
    i0 = DNCHUNK - 2
    idx_wait(i0, 0)
    idx_start(i0 + 1, 1)
    scat(0)
    idx_wait(i0 + 1, 1)
    scat(1)
    plsc.subcore_barrier()

    wch2 = N // NW // 25
    for t in range(25):
        r0 = wid * (N // NW) + t * wch2
        pltpu.sync_copy(shared.at[pl.ds(r0, wch2), :],
                        sbufd.at[pl.ds(0, wch2), :])

        def srow(j, _):
            v = sbufd[j]
            pos = v > 0
            safe = jnp.where(pos, v, 1.0)
            d1b[j] = jnp.where(pos, _rsqrt_nr(safe), 0.0)
            d2b[j] = jnp.where(pos, 1.0 / safe, 0.0)
            return ()

        lax.fori_loop(0, wch2, srow, ())
        pltpu.sync_copy(d1b.at[pl.ds(0, wch2), :], d1e.at[pl.ds(r0, wch2), :])
        pltpu.sync_copy(d2b.at[pl.ds(0, wch2), :], d2e.at[pl.ds(r0, wch2), :])


def _degree(dst, ones16):
    k = pl.kernel(
        _deg_kernel,
        out_type=(jax.ShapeDtypeStruct((N, D), _f32),
                  jax.ShapeDtypeStruct((N, D), _f32)),
        mesh=_sc_mesh(),
        scratch_types=dict(
            shared=pltpu.VMEM_SHARED((N, D), _f32),
            didx0=pltpu.VMEM((DCH,), jnp.int32),
            didx1=pltpu.VMEM((DCH,), jnp.int32),
            sbufd=pltpu.VMEM((WCH, D), _f32),
            d1b=pltpu.VMEM((WCH, D), _f32),
            d2b=pltpu.VMEM((WCH, D), _f32),
            obuf=pltpu.VMEM((DCH, D), _f32),
            is0=pltpu.SemaphoreType.DMA,
            is1=pltpu.SemaphoreType.DMA,
        ),
        compiler_params=_SC_PARAMS,
        name="sc_deg",
    )
    return k(dst, ones16)


def _edge_sweep(tbl, eidx, accum, ebs, rbs, gss, sss, iss, c, s, rb0):
    def zb(j, _):
        rb0[j] = jnp.zeros((16,), _f32)
        return ()

    lax.fori_loop(0, WCH, zb, ())
    for t in range(NPT // WCH):
        pltpu.sync_copy(rb0.at[pl.ds(0, WCH), :],
                        accum.at[pl.ds(s * NPT + t * WCH, WCH), :])
    plsc.subcore_barrier()

    def idx_start(i, p):
        base = s * EPT + i * CH
        return pltpu.async_copy(eidx.at[:, pl.ds(base, CH)], ebs[p], iss[p])

    def gather_start(p):
        return pltpu.async_copy(tbl.at[c].at[ebs[p].at[0]], rbs[p], gss[p])

    def gather_wait(p):
        pltpu.make_async_copy(tbl.at[c].at[ebs[p].at[0]], rbs[p], gss[p]).wait()

    def scat_start(p):
        return pltpu.async_copy(rbs[p], accum.at[ebs[p].at[1]], sss[p], add=True)

    def scat_wait(p):
        pltpu.make_async_copy(rbs[p], accum.at[ebs[p].at[1]], sss[p]).wait()

    idx_start(0, 0).wait()
    gather_start(0)
    idx_start(1, 1).wait()
    gather_wait(0)
    scat_start(0)
    gather_start(1)

    def chunk_body(i, p):
        scat_wait(p)
        idesc = idx_start(i, p)
        gather_wait(1 - p)
        scat_start(1 - p)
        idesc.wait()
        gather_start(p)
        return ()

    def pair(k, _):
        i0 = 2 + 2 * k
        chunk_body(i0, 0)
        chunk_body(i0 + 1, 1)
        return ()

    lax.fori_loop(0, (NCHUNK - 2) // 2, pair, ())
    if (NCHUNK - 2) % 2 == 1:
        chunk_body(NCHUNK - 1, 0)
        last = 0
    else:
        last = 1
    gather_wait(last)
    scat_start(last)
    scat_wait(1 - last)
    scat_wait(last)
    plsc.subcore_barrier()


def _l1_body(eidx, x32, d1e, d2e, g0, g1, g2, u128, accum,
             eb0, eb1, rb0, rb1, gs0, gs1, ss0, ss1, is0, is1):
    c = lax.axis_index("c")
    s = lax.axis_index("s")
    ebs, rbs = (eb0, eb1), (rb0, rb1)
    gss, sss, iss = (gs0, gs1), (ss0, ss1), (is0, is1)

    pltpu.sync_copy(x32.at[pl.ds(s * NPT, NPT), :],
                    u128.at[pl.ds(s * NPT, NPT), pl.ds(0, 32)])
    for t in range(NPT // WCH):
        r0 = s * NPT + t * WCH
        pltpu.sync_copy(x32.at[pl.ds(r0, WCH), pl.ds(c * D, D)],
                        rb0.at[pl.ds(0, WCH), :])
        pltpu.sync_copy(d1e.at[pl.ds(r0, WCH), :],
                        rb0.at[pl.ds(WCH, WCH), :])

        def crow(j, _):
            rb0[j] = rb0[j] * rb0[WCH + j]
            return ()

        lax.fori_loop(0, WCH, crow, ())
        pltpu.sync_copy(rb0.at[pl.ds(0, WCH), :],
                        g0.at[c].at[pl.ds(r0, WCH), :])
    plsc.subcore_barrier()

    for (rnd, tbl, gout) in ((1, g0, g1), (2, g1, g2), (3, g2, None)):
        _edge_sweep(tbl, eidx, accum, ebs, rbs, gss, sss, iss, c, s, rb0)
        for t in range(NPT // WCH):
            r0 = s * NPT + t * WCH
            pltpu.sync_copy(accum.at[pl.ds(r0, WCH), :],
                            rb0.at[pl.ds(0, WCH), :])
            pltpu.sync_copy(d2e.at[pl.ds(r0, WCH), :],
                            rb0.at[pl.ds(WCH, WCH), :])
            pltpu.sync_copy(d1e.at[pl.ds(r0, WCH), :],
                            rb0.at[pl.ds(2 * WCH, WCH), :])

            def wrow(j, _):
                a = rb0[j]
                if gout is not None:
                    rb0[j] = a * rb0[WCH + j]
                rb0[2 * WCH + j] = a * rb0[2 * WCH + j]
                return ()

            lax.fori_loop(0, WCH, wrow, ())
            if gout is not None:
                pltpu.sync_copy(rb0.at[pl.ds(0, WCH), :],
                                gout.at[c].at[pl.ds(r0, WCH), :])
            pltpu.sync_copy(
                rb0.at[pl.ds(2 * WCH, WCH), :],
                u128.at[pl.ds(r0, WCH), pl.ds(32 * rnd + c * D, D)])
        plsc.subcore_barrier()


def _l2_body(eidx, Z, d1e, d2e, g4, w2, w3, o32, accum,
             eb0, eb1, rb0, rb1, gs0, gs1, ss0, ss1, is0, is1):
    c = lax.axis_index("c")
    s = lax.axis_index("s")
    ebs, rbs = (eb0, eb1), (rb0, rb1)
    gss, sss, iss = (gs0, gs1), (ss0, ss1), (is0, is1)

    for t in range(NPT // WCH):
        r0 = s * NPT + t * WCH
        pltpu.sync_copy(Z.at[pl.ds(r0, WCH), pl.ds(96 + c * D, D)],
                        rb0.at[pl.ds(0, WCH), :])
        pltpu.sync_copy(d1e.at[pl.ds(r0, WCH), :],
                        rb0.at[pl.ds(WCH, WCH), :])

        def crow(j, _):
            rb0[j] = rb0[j] * rb0[WCH + j]
            return ()

        lax.fori_loop(0, WCH, crow, ())
        pltpu.sync_copy(rb0.at[pl.ds(0, WCH), :],
                        g4.at[c].at[pl.ds(r0, WCH), :])
    plsc.subcore_barrier()

    for (rnd, tbl, gout, zc) in ((4, g4, w2, 64), (5, w2, w3, 32),
                                 (6, w3, None, 0)):
        _edge_sweep(tbl, eidx, accum, ebs, rbs, gss, sss, iss, c, s, rb0)
        for t in range(NPT // WCH):
            r0 = s * NPT + t * WCH
            pltpu.sync_copy(accum.at[pl.ds(r0, WCH), :],
                            rb0.at[pl.ds(0, WCH), :])
            pltpu.sync_copy(Z.at[pl.ds(r0, WCH), pl.ds(zc + c * D, D)],
                            rb0.at[pl.ds(WCH, WCH), :])
            pltpu.sync_copy(d1e.at[pl.ds(r0, WCH), :],
                            rb0.at[pl.ds(2 * WCH, WCH), :])
            if gout is not None:
                pltpu.sync_copy(d2e.at[pl.ds(r0, WCH), :],
                                rb1.at[pl.ds(0, WCH), :])

                def wrow1(j, _):
                    rb0[j] = (rb0[j] * rb1[j]
                              + rb0[WCH + j] * rb0[2 * WCH + j])
                    return ()

                lax.fori_loop(0, WCH, wrow1, ())
                pltpu.sync_copy(rb0.at[pl.ds(0, WCH), :],
                                gout.at[c].at[pl.ds(r0, WCH), :])
            else:
                def frow(j, _):
                    v = rb0[WCH + j] + rb0[j] * rb0[2 * WCH + j]
                    rb0[j] = jnp.where(v > 0, v, 0.25 * v)
                    return ()

                lax.fori_loop(0, WCH, frow, ())
                pltpu.sync_copy(rb0.at[pl.ds(0, WCH), :],
                                o32.at[pl.ds(r0, WCH), pl.ds(c * D, D)])
        plsc.subcore_barrier()


_PROP_SCRATCH = dict(
    accum=pltpu.VMEM_SHARED((N, D), _f32),
    eb0=pltpu.VMEM((2, CH), jnp.int32),
    eb1=pltpu.VMEM((2, CH), jnp.int32),
    rb0=pltpu.VMEM((CH, D), _f32),
    rb1=pltpu.VMEM((CH, D), _f32),
    gs0=pltpu.SemaphoreType.DMA,
    gs1=pltpu.SemaphoreType.DMA,
    ss0=pltpu.SemaphoreType.DMA,
    ss1=pltpu.SemaphoreType.DMA,
    is0=pltpu.SemaphoreType.DMA,
    is1=pltpu.SemaphoreType.DMA,
)


def _run_l1(eidx, x32, d1e, d2e):
    gt = jax.ShapeDtypeStruct((NC, N, D), _f32)
    k = pl.kernel(
        _l1_body,
        out_type=(gt, gt, gt, jax.ShapeDtypeStruct((N, 128), _f32)),
        mesh=_sc_mesh(),
        scratch_types=dict(_PROP_SCRATCH),
        compiler_params=_SC_PARAMS,
        name="sc_prop_l1",
    )
    return k(eidx, x32, d1e, d2e)


def _run_l2(eidx, Z, d1e, d2e):
    gt = jax.ShapeDtypeStruct((NC, N, D), _f32)
    k = pl.kernel(
        _l2_body,
        out_type=(gt, gt, gt, jax.ShapeDtypeStruct((N, 32), _f32)),
        mesh=_sc_mesh(),
        scratch_types=dict(_PROP_SCRATCH),
        compiler_params=_SC_PARAMS,
        name="sc_prop_l2",
    )
    return k(eidx, Z, d1e, d2e)


BM = 2000


def _main_body(u_ref, w1t_ref, b1_ref, w2r_ref, b2p_ref, z_ref):
    y = jnp.dot(u_ref[...], w1t_ref[...],
                preferred_element_type=_f32) + b1_ref[...]
    h = jnp.where(y > 0, y, 0.25 * y)
    z_ref[...] = jnp.dot(h, w2r_ref[...],
                         preferred_element_type=_f32) + b2p_ref[...]


def _tc_main(u128, W1, b1, W2, b2):
    w1t = W1.T
    w2r = jnp.concatenate(
        [W2[:, 64 * j:64 * (j + 1)].T for j in range(4)], axis=1)
    b2p = jnp.concatenate([b2, jnp.zeros((96,), _f32)]).reshape(1, 128)
    return pl.pallas_call(
        _main_body,
        grid=(N // BM,),
        in_specs=[
            pl.BlockSpec((BM, 128), lambda i: (i, 0)),
            pl.BlockSpec((128, 64), lambda i: (0, 0)),
            pl.BlockSpec((1, 64), lambda i: (0, 0)),
            pl.BlockSpec((64, 128), lambda i: (0, 0)),
            pl.BlockSpec((1, 128), lambda i: (0, 0)),
        ],
        out_specs=pl.BlockSpec((BM, 128), lambda i: (i, 0)),
        out_shape=jax.ShapeDtypeStruct((N, 128), _f32),
    )(u128, w1t, b1.reshape(1, 64), w2r, b2p)


def kernel(category, noise, edge_index, W1, b1, W2, b2):
    eidx = edge_index.astype(jnp.int32)
    dst = eidx[1]
    ones16 = jnp.ones((DCH, D), _f32)
    x32 = jnp.concatenate([category, noise], axis=1)

    d1e, d2e = _degree(dst, ones16)

    _, _, _, u128 = _run_l1(eidx, x32, d1e, d2e)
    Z = _tc_main(u128, W1, b1, W2, b2)
    _, _, _, o32 = _run_l2(eidx, Z, d1e, d2e)
    return o32


# speedup vs baseline: 1.2868x; 1.2868x over previous
import functools

import jax
import jax.numpy as jnp
from jax import lax
from jax.experimental import pallas as pl
from jax.experimental.pallas import tpu as pltpu
from jax.experimental.pallas import tpu_sc as plsc

N = 100000
E = 1600000
NC = 2
NS = 16
NW = NC * NS
D = 16
CH = 800
NCHUNK = E // NS // CH
EPT = E // NS
WCH = 250
NPT = N // NS
DCH = 1000
DNCHUNK = EPT // DCH

_f32 = jnp.float32


def _sc_mesh():
    return plsc.VectorSubcoreMesh(
        core_axis_name="c", subcore_axis_name="s", num_cores=NC, num_subcores=NS
    )


_SC_PARAMS = pltpu.CompilerParams(
    use_tc_tiling_on_sc=False, needs_layout_passes=False)


def _rsqrt_nr(v):
    i = plsc.bitcast(v, jnp.int32)
    y = plsc.bitcast(jnp.int32(0x5F3759DF) - (i >> 1), _f32)
    for _ in range(4):
        y = y * (1.5 - 0.5 * v * y * y)
    return y


def _deg_kernel(dst, ones16, d1e, d2e, shared, didx0, didx1,
                sbufd, d1b, d2b, obuf, is0, is1):
    c = lax.axis_index("c")
    s = lax.axis_index("s")
    wid = c * NS + s
    pltpu.sync_copy(ones16, obuf)

    def zb(j, _):
        d1b[j] = jnp.zeros((16,), _f32)
        return ()

    lax.fori_loop(0, WCH, zb, ())
    for t in range(NPT // WCH):
        pltpu.sync_copy(d1b, shared.at[pl.ds(s * NPT + t * WCH, WCH), :])
    plsc.subcore_barrier()

    dbs = (didx0, didx1)
    iss = (is0, is1)

    def idx_start(i, p):
        base = s * EPT + i * DCH
        return pltpu.async_copy(dst.at[pl.ds(base, DCH)], dbs[p], iss[p])

    def idx_wait(i, p):
        base = s * EPT + i * DCH
        pltpu.make_async_copy(dst.at[pl.ds(base, DCH)], dbs[p], iss[p]).wait()

    def scat(p):
        pltpu.sync_copy(obuf, shared.at[dbs[p]], add=True)

    idx_start(0, 0)

    def pair(k, _):
        i0 = 2 * k
        idx_wait(i0, 0)
        idx_start(i0 + 1, 1)
        scat(0)
        idx_wait(i0 + 1, 1)
        idx_start(i0 + 2, 0)
        scat(1)
        return ()

    assert DNCHUNK % 2 == 0
    lax.fori_loop(0, DNCHUNK // 2 - 1, pair, ())
    i0 = DNCHUNK - 2
    idx_wait(i0, 0)
    idx_start(i0 + 1, 1)
    scat(0)
    idx_wait(i0 + 1, 1)
    scat(1)
    plsc.subcore_barrier()

    wch2 = N // NW // 25
    for t in range(25):
        r0 = wid * (N // NW) + t * wch2
        pltpu.sync_copy(shared.at[pl.ds(r0, wch2), :],
                        sbufd.at[pl.ds(0, wch2), :])

        def srow(j, _):
            v = sbufd[j]
            pos = v > 0
            safe = jnp.where(pos, v, 1.0)
            d1b[j] = jnp.where(pos, _rsqrt_nr(safe), 0.0)
            d2b[j] = jnp.where(pos, 1.0 / safe, 0.0)
            return ()

        lax.fori_loop(0, wch2, srow, ())
        pltpu.sync_copy(d1b.at[pl.ds(0, wch2), :], d1e.at[pl.ds(r0, wch2), :])
        pltpu.sync_copy(d2b.at[pl.ds(0, wch2), :], d2e.at[pl.ds(r0, wch2), :])


def _degree(dst, ones16):
    k = pl.kernel(
        _deg_kernel,
        out_type=(jax.ShapeDtypeStruct((N, D), _f32),
                  jax.ShapeDtypeStruct((N, D), _f32)),
        mesh=_sc_mesh(),
        scratch_types=dict(
            shared=pltpu.VMEM_SHARED((N, D), _f32),
            didx0=pltpu.VMEM((DCH,), jnp.int32),
            didx1=pltpu.VMEM((DCH,), jnp.int32),
            sbufd=pltpu.VMEM((WCH, D), _f32),
            d1b=pltpu.VMEM((WCH, D), _f32),
            d2b=pltpu.VMEM((WCH, D), _f32),
            obuf=pltpu.VMEM((DCH, D), _f32),
            is0=pltpu.SemaphoreType.DMA,
            is1=pltpu.SemaphoreType.DMA,
        ),
        compiler_params=_SC_PARAMS,
        name="sc_deg",
    )
    return k(dst, ones16)


def _edge_sweep(tbl, eidx, accum, ebs, rbs, gss, sss, iss, c, s, rb0):
    def zb(j, _):
        rb0[j] = jnp.zeros((16,), _f32)
        return ()

    lax.fori_loop(0, WCH, zb, ())
    for t in range(NPT // WCH):
        pltpu.sync_copy(rb0.at[pl.ds(0, WCH), :],
                        accum.at[pl.ds(s * NPT + t * WCH, WCH), :])
    plsc.subcore_barrier()

    def idx_start(i, p):
        base = s * EPT + i * CH
        return pltpu.async_copy(eidx.at[:, pl.ds(base, CH)], ebs[p], iss[p])

    def gather_start(p):
        return pltpu.async_copy(tbl.at[c].at[ebs[p].at[0]], rbs[p], gss[p])

    def gather_wait(p):
        pltpu.make_async_copy(tbl.at[c].at[ebs[p].at[0]], rbs[p], gss[p]).wait()

    def scat_start(p):
        return pltpu.async_copy(rbs[p], accum.at[ebs[p].at[1]], sss[p], add=True)

    def scat_wait(p):
        pltpu.make_async_copy(rbs[p], accum.at[ebs[p].at[1]], sss[p]).wait()

    idx_start(0, 0).wait()
    gather_start(0)
    idx_start(1, 1).wait()
    gather_wait(0)
    scat_start(0)
    gather_start(1)

    def chunk_body(i, p):
        scat_wait(p)
        idesc = idx_start(i, p)
        gather_wait(1 - p)
        scat_start(1 - p)
        idesc.wait()
        gather_start(p)
        return ()

    def pair(k, _):
        i0 = 2 + 2 * k
        chunk_body(i0, 0)
        chunk_body(i0 + 1, 1)
        return ()

    lax.fori_loop(0, (NCHUNK - 2) // 2, pair, ())
    if (NCHUNK - 2) % 2 == 1:
        chunk_body(NCHUNK - 1, 0)
        last = 0
    else:
        last = 1
    gather_wait(last)
    scat_start(last)
    scat_wait(1 - last)
    scat_wait(last)
    plsc.subcore_barrier()


def _l1_body(eidx, x32, d1e, d2e, g0, g1, g2, u128, accum,
             eb0, eb1, rb0, rb1, gs0, gs1, ss0, ss1, is0, is1):
    c = lax.axis_index("c")
    s = lax.axis_index("s")
    ebs, rbs = (eb0, eb1), (rb0, rb1)
    gss, sss, iss = (gs0, gs1), (ss0, ss1), (is0, is1)

    for t in range(NPT // WCH):
        r0 = s * NPT + t * WCH
        for half in range(2):
            pltpu.sync_copy(x32.at[pl.ds(r0, WCH), pl.ds(16 * half, 16)],
                            rb1.at[pl.ds(0, WCH), :])
            pltpu.sync_copy(rb1.at[pl.ds(0, WCH), :],
                            u128.at[pl.ds(r0, WCH), pl.ds(16 * half, 16)])
        pltpu.sync_copy(x32.at[pl.ds(r0, WCH), pl.ds(c * D, D)],
                        rb0.at[pl.ds(0, WCH), :])
        pltpu.sync_copy(d1e.at[pl.ds(r0, WCH), :],
                        rb0.at[pl.ds(WCH, WCH), :])

        def crow(j, _):
            rb0[j] = rb0[j] * rb0[WCH + j]
            return ()

        lax.fori_loop(0, WCH, crow, ())
        pltpu.sync_copy(rb0.at[pl.ds(0, WCH), :],
                        g0.at[c].at[pl.ds(r0, WCH), :])
    plsc.subcore_barrier()

    for (rnd, tbl, gout) in ((1, g0, g1), (2, g1, g2), (3, g2, None)):
        _edge_sweep(tbl, eidx, accum, ebs, rbs, gss, sss, iss, c, s, rb0)
        for t in range(NPT // WCH):
            r0 = s * NPT + t * WCH
            pltpu.sync_copy(accum.at[pl.ds(r0, WCH), :],
                            rb0.at[pl.ds(0, WCH), :])
            pltpu.sync_copy(d2e.at[pl.ds(r0, WCH), :],
                            rb0.at[pl.ds(WCH, WCH), :])
            pltpu.sync_copy(d1e.at[pl.ds(r0, WCH), :],
                            rb0.at[pl.ds(2 * WCH, WCH), :])

            def wrow(j, _):
                a = rb0[j]
                if gout is not None:
                    rb0[j] = a * rb0[WCH + j]
                rb0[2 * WCH + j] = a * rb0[2 * WCH + j]
                return ()

            lax.fori_loop(0, WCH, wrow, ())
            if gout is not None:
                pltpu.sync_copy(rb0.at[pl.ds(0, WCH), :],
                                gout.at[c].at[pl.ds(r0, WCH), :])
            pltpu.sync_copy(
                rb0.at[pl.ds(2 * WCH, WCH), :],
                u128.at[pl.ds(r0, WCH), pl.ds(32 * rnd + c * D, D)])
        plsc.subcore_barrier()


def _l2_body(eidx, Z, d1e, d2e, g4, w2, w3, o32, accum,
             eb0, eb1, rb0, rb1, gs0, gs1, ss0, ss1, is0, is1):
    c = lax.axis_index("c")
    s = lax.axis_index("s")
    ebs, rbs = (eb0, eb1), (rb0, rb1)
    gss, sss, iss = (gs0, gs1), (ss0, ss1), (is0, is1)

    for t in range(NPT // WCH):
        r0 = s * NPT + t * WCH
        pltpu.sync_copy(Z.at[pl.ds(r0, WCH), pl.ds(96 + c * D, D)],
                        rb0.at[pl.ds(0, WCH), :])
        pltpu.sync_copy(d1e.at[pl.ds(r0, WCH), :],
                        rb0.at[pl.ds(WCH, WCH), :])

        def crow(j, _):
            rb0[j] = rb0[j] * rb0[WCH + j]
            return ()

        lax.fori_loop(0, WCH, crow, ())
        pltpu.sync_copy(rb0.at[pl.ds(0, WCH), :],
                        g4.at[c].at[pl.ds(r0, WCH), :])
    plsc.subcore_barrier()

    for (rnd, tbl, gout, zc) in ((4, g4, w2, 64), (5, w2, w3, 32),
                                 (6, w3, None, 0)):
        _edge_sweep(tbl, eidx, accum, ebs, rbs, gss, sss, iss, c, s, rb0)
        for t in range(NPT // WCH):
            r0 = s * NPT + t * WCH
            pltpu.sync_copy(accum.at[pl.ds(r0, WCH), :],
                            rb0.at[pl.ds(0, WCH), :])
            pltpu.sync_copy(Z.at[pl.ds(r0, WCH), pl.ds(zc + c * D, D)],
                            rb0.at[pl.ds(WCH, WCH), :])
            pltpu.sync_copy(d1e.at[pl.ds(r0, WCH), :],
                            rb0.at[pl.ds(2 * WCH, WCH), :])
            if gout is not None:
                pltpu.sync_copy(d2e.at[pl.ds(r0, WCH), :],
                                rb1.at[pl.ds(0, WCH), :])

                def wrow1(j, _):
                    rb0[j] = (rb0[j] * rb1[j]
                              + rb0[WCH + j] * rb0[2 * WCH + j])
                    return ()

                lax.fori_loop(0, WCH, wrow1, ())
                pltpu.sync_copy(rb0.at[pl.ds(0, WCH), :],
                                gout.at[c].at[pl.ds(r0, WCH), :])
            else:
                def frow(j, _):
                    v = rb0[WCH + j] + rb0[j] * rb0[2 * WCH + j]
                    rb0[j] = jnp.where(v > 0, v, 0.25 * v)
                    return ()

                lax.fori_loop(0, WCH, frow, ())
                pltpu.sync_copy(rb0.at[pl.ds(0, WCH), :],
                                o32.at[pl.ds(r0, WCH), pl.ds(c * D, D)])
        plsc.subcore_barrier()


_PROP_SCRATCH = dict(
    accum=pltpu.VMEM_SHARED((N, D), _f32),
    eb0=pltpu.VMEM((2, CH), jnp.int32),
    eb1=pltpu.VMEM((2, CH), jnp.int32),
    rb0=pltpu.VMEM((CH, D), _f32),
    rb1=pltpu.VMEM((CH, D), _f32),
    gs0=pltpu.SemaphoreType.DMA,
    gs1=pltpu.SemaphoreType.DMA,
    ss0=pltpu.SemaphoreType.DMA,
    ss1=pltpu.SemaphoreType.DMA,
    is0=pltpu.SemaphoreType.DMA,
    is1=pltpu.SemaphoreType.DMA,
)


def _run_l1(eidx, x32, d1e, d2e):
    gt = jax.ShapeDtypeStruct((NC, N, D), _f32)
    k = pl.kernel(
        _l1_body,
        out_type=(gt, gt, gt, jax.ShapeDtypeStruct((N, 128), _f32)),
        mesh=_sc_mesh(),
        scratch_types=dict(_PROP_SCRATCH),
        compiler_params=_SC_PARAMS,
        name="sc_prop_l1",
    )
    return k(eidx, x32, d1e, d2e)


def _run_l2(eidx, Z, d1e, d2e):
    gt = jax.ShapeDtypeStruct((NC, N, D), _f32)
    k = pl.kernel(
        _l2_body,
        out_type=(gt, gt, gt, jax.ShapeDtypeStruct((N, 32), _f32)),
        mesh=_sc_mesh(),
        scratch_types=dict(_PROP_SCRATCH),
        compiler_params=_SC_PARAMS,
        name="sc_prop_l2",
    )
    return k(eidx, Z, d1e, d2e)


BM = 2000


def _main_body(u_ref, w1t_ref, b1_ref, w2r_ref, b2p_ref, z_ref):
    y = jnp.dot(u_ref[...], w1t_ref[...],
                preferred_element_type=_f32) + b1_ref[...]
    h = jnp.where(y > 0, y, 0.25 * y)
    z_ref[...] = jnp.dot(h, w2r_ref[...],
                         preferred_element_type=_f32) + b2p_ref[...]


def _tc_main(u128, W1, b1, W2, b2):
    w1t = W1.T
    w2r = jnp.concatenate(
        [W2[:, 64 * j:64 * (j + 1)].T for j in range(4)], axis=1)
    b2p = jnp.concatenate([b2, jnp.zeros((96,), _f32)]).reshape(1, 128)
    return pl.pallas_call(
        _main_body,
        grid=(N // BM,),
        in_specs=[
            pl.BlockSpec((BM, 128), lambda i: (i, 0)),
            pl.BlockSpec((128, 64), lambda i: (0, 0)),
            pl.BlockSpec((1, 64), lambda i: (0, 0)),
            pl.BlockSpec((64, 128), lambda i: (0, 0)),
            pl.BlockSpec((1, 128), lambda i: (0, 0)),
        ],
        out_specs=pl.BlockSpec((BM, 128), lambda i: (i, 0)),
        out_shape=jax.ShapeDtypeStruct((N, 128), _f32),
    )(u128, w1t, b1.reshape(1, 64), w2r, b2p)


def kernel(category, noise, edge_index, W1, b1, W2, b2):
    eidx = edge_index.astype(jnp.int32)
    dst = eidx[1]
    ones16 = jnp.ones((DCH, D), _f32)
    x32 = jnp.concatenate([category, noise], axis=1)

    d1e, d2e = _degree(dst, ones16)

    _, _, _, u128 = _run_l1(eidx, x32, d1e, d2e)
    Z = _tc_main(u128, W1, b1, W2, b2)
    _, _, _, o32 = _run_l2(eidx, Z, d1e, d2e)
    return o32


# speedup vs baseline: 1.3519x; 1.0506x over previous
import functools

import jax
import jax.numpy as jnp
from jax import lax
from jax.experimental import pallas as pl
from jax.experimental.pallas import tpu as pltpu
from jax.experimental.pallas import tpu_sc as plsc

N = 100000
E = 1600000
NC = 2
NS = 16
NW = NC * NS
D = 16
CH = 800
NCHUNK = E // NS // CH
EPT = E // NS
WCH = 250
NPT = N // NS
DCH = 1000
DNCHUNK = EPT // DCH

_f32 = jnp.float32


def _sc_mesh():
    return plsc.VectorSubcoreMesh(
        core_axis_name="c", subcore_axis_name="s", num_cores=NC, num_subcores=NS
    )


_SC_PARAMS = pltpu.CompilerParams(
    use_tc_tiling_on_sc=False, needs_layout_passes=False)


def _rsqrt_nr(v):
    i = plsc.bitcast(v, jnp.int32)
    y = plsc.bitcast(jnp.int32(0x5F3759DF) - (i >> 1), _f32)
    for _ in range(4):
        y = y * (1.5 - 0.5 * v * y * y)
    return y


def _deg_kernel(dst, ones16, d1e, shared, didx0, didx1,
                sbufd, d1b, obuf, is0, is1):
    c = lax.axis_index("c")
    s = lax.axis_index("s")
    wid = c * NS + s
    pltpu.sync_copy(ones16, obuf)

    def zb(j, _):
        d1b[j] = jnp.zeros((16,), _f32)
        return ()

    lax.fori_loop(0, WCH, zb, ())
    for t in range(NPT // WCH):
        pltpu.sync_copy(d1b, shared.at[pl.ds(s * NPT + t * WCH, WCH), :])
    plsc.subcore_barrier()

    dbs = (didx0, didx1)
    iss = (is0, is1)

    def idx_start(i, p):
        base = s * EPT + i * DCH
        return pltpu.async_copy(dst.at[pl.ds(base, DCH)], dbs[p], iss[p])

    def idx_wait(i, p):
        base = s * EPT + i * DCH
        pltpu.make_async_copy(dst.at[pl.ds(base, DCH)], dbs[p], iss[p]).wait()

    def scat(p):
        pltpu.sync_copy(obuf, shared.at[dbs[p]], add=True)

    idx_start(0, 0)

    def pair(k, _):
        i0 = 2 * k
        idx_wait(i0, 0)
        idx_start(i0 + 1, 1)
        scat(0)
        idx_wait(i0 + 1, 1)
        idx_start(i0 + 2, 0)
        scat(1)
        return ()

    assert DNCHUNK % 2 == 0
    lax.fori_loop(0, DNCHUNK // 2 - 1, pair, ())
    i0 = DNCHUNK - 2
    idx_wait(i0, 0)
    idx_start(i0 + 1, 1)
    scat(0)
    idx_wait(i0 + 1, 1)
    scat(1)
    plsc.subcore_barrier()

    wch2 = N // NW // 25
    for t in range(25):
        r0 = wid * (N // NW) + t * wch2
        pltpu.sync_copy(shared.at[pl.ds(r0, wch2), :],
                        sbufd.at[pl.ds(0, wch2), :])

        def srow(j, _):
            v = sbufd[j]
            pos = v > 0
            safe = jnp.where(pos, v, 1.0)
            d1b[j] = jnp.where(pos, _rsqrt_nr(safe), 0.0)
            return ()

        lax.fori_loop(0, wch2, srow, ())
        pltpu.sync_copy(d1b.at[pl.ds(0, wch2), :], d1e.at[pl.ds(r0, wch2), :])


def _degree(dst, ones16):
    k = pl.kernel(
        _deg_kernel,
        out_type=jax.ShapeDtypeStruct((N, D), _f32),
        mesh=_sc_mesh(),
        scratch_types=dict(
            shared=pltpu.VMEM_SHARED((N, D), _f32),
            didx0=pltpu.VMEM((DCH,), jnp.int32),
            didx1=pltpu.VMEM((DCH,), jnp.int32),
            sbufd=pltpu.VMEM((WCH, D), _f32),
            d1b=pltpu.VMEM((WCH, D), _f32),
            obuf=pltpu.VMEM((DCH, D), _f32),
            is0=pltpu.SemaphoreType.DMA,
            is1=pltpu.SemaphoreType.DMA,
        ),
        compiler_params=_SC_PARAMS,
        name="sc_deg",
    )
    return k(dst, ones16)


def _edge_sweep(tbl, eidx, accum, ebs, rbs, gss, sss, iss, c, s, rb0):
    def zb(j, _):
        rb0[j] = jnp.zeros((16,), _f32)
        return ()

    lax.fori_loop(0, WCH, zb, ())
    for t in range(NPT // WCH):
        pltpu.sync_copy(rb0.at[pl.ds(0, WCH), :],
                        accum.at[pl.ds(s * NPT + t * WCH, WCH), :])
    plsc.subcore_barrier()

    def idx_start(i, p):
        base = s * EPT + i * CH
        return pltpu.async_copy(eidx.at[:, pl.ds(base, CH)], ebs[p], iss[p])

    def gather_start(p):
        return pltpu.async_copy(tbl.at[c].at[ebs[p].at[0]], rbs[p], gss[p])

    def gather_wait(p):
        pltpu.make_async_copy(tbl.at[c].at[ebs[p].at[0]], rbs[p], gss[p]).wait()

    def scat_start(p):
        return pltpu.async_copy(rbs[p], accum.at[ebs[p].at[1]], sss[p], add=True)

    def scat_wait(p):
        pltpu.make_async_copy(rbs[p], accum.at[ebs[p].at[1]], sss[p]).wait()

    idx_start(0, 0).wait()
    gather_start(0)
    idx_start(1, 1).wait()
    gather_wait(0)
    scat_start(0)
    gather_start(1)

    def chunk_body(i, p):
        scat_wait(p)
        idesc = idx_start(i, p)
        gather_wait(1 - p)
        scat_start(1 - p)
        idesc.wait()
        gather_start(p)
        return ()

    def pair(k, _):
        i0 = 2 + 2 * k
        chunk_body(i0, 0)
        chunk_body(i0 + 1, 1)
        return ()

    lax.fori_loop(0, (NCHUNK - 2) // 2, pair, ())
    if (NCHUNK - 2) % 2 == 1:
        chunk_body(NCHUNK - 1, 0)
        last = 0
    else:
        last = 1
    gather_wait(last)
    scat_start(last)
    scat_wait(1 - last)
    scat_wait(last)
    plsc.subcore_barrier()


def _l1_body(eidx, x32, d1e, g0, g1, g2, u128, accum,
             eb0, eb1, rb0, rb1, gs0, gs1, ss0, ss1, is0, is1):
    c = lax.axis_index("c")
    s = lax.axis_index("s")
    ebs, rbs = (eb0, eb1), (rb0, rb1)
    gss, sss, iss = (gs0, gs1), (ss0, ss1), (is0, is1)

    for t in range(NPT // WCH):
        r0 = s * NPT + t * WCH
        for half in range(2):
            pltpu.sync_copy(x32.at[pl.ds(r0, WCH), pl.ds(16 * half, 16)],
                            rb1.at[pl.ds(0, WCH), :])
            pltpu.sync_copy(rb1.at[pl.ds(0, WCH), :],
                            u128.at[pl.ds(r0, WCH), pl.ds(16 * half, 16)])
        pltpu.sync_copy(x32.at[pl.ds(r0, WCH), pl.ds(c * D, D)],
                        rb0.at[pl.ds(0, WCH), :])
        pltpu.sync_copy(d1e.at[pl.ds(r0, WCH), :],
                        rb0.at[pl.ds(WCH, WCH), :])

        def crow(j, _):
            rb0[j] = rb0[j] * rb0[WCH + j]
            return ()

        lax.fori_loop(0, WCH, crow, ())
        pltpu.sync_copy(rb0.at[pl.ds(0, WCH), :],
                        g0.at[c].at[pl.ds(r0, WCH), :])
    plsc.subcore_barrier()

    for (rnd, tbl, gout) in ((1, g0, g1), (2, g1, g2), (3, g2, None)):
        _edge_sweep(tbl, eidx, accum, ebs, rbs, gss, sss, iss, c, s, rb0)
        nt = NPT // WCH

        def wb_writes(t):
            A = rbs[t % 2]
            r0 = s * NPT + t * WCH
            outs = [(A.at[pl.ds(2 * WCH, WCH), :],
                     u128.at[pl.ds(r0, WCH), pl.ds(32 * rnd + c * D, D)])]
            if gout is not None:
                outs.append((A.at[pl.ds(0, WCH), :],
                             gout.at[c].at[pl.ds(r0, WCH), :]))
            return outs

        for t in range(nt):
            A = rbs[t % 2]
            sem = iss[t % 2]
            r0 = s * NPT + t * WCH
            if t >= 2:
                for (sr, dr) in wb_writes(t - 2):
                    pltpu.make_async_copy(sr, dr, sem).wait()
            pltpu.sync_copy(accum.at[pl.ds(r0, WCH), :],
                            A.at[pl.ds(0, WCH), :])
            pltpu.sync_copy(d1e.at[pl.ds(r0, WCH), :],
                            A.at[pl.ds(WCH, WCH), :])

            def wrow(j, _):
                a = A[j] * A[WCH + j]
                A[2 * WCH + j] = a
                if gout is not None:
                    A[j] = a * A[WCH + j]
                return ()

            lax.fori_loop(0, WCH, wrow, ())
            for (sr, dr) in wb_writes(t):
                pltpu.async_copy(sr, dr, sem)
        for t in (nt - 2, nt - 1):
            for (sr, dr) in wb_writes(t):
                pltpu.make_async_copy(sr, dr, iss[t % 2]).wait()
        plsc.subcore_barrier()


def _l2_body(eidx, Z, d1e, g4, w2, w3, o32, accum,
             eb0, eb1, rb0, rb1, gs0, gs1, ss0, ss1, is0, is1):
    c = lax.axis_index("c")
    s = lax.axis_index("s")
    ebs, rbs = (eb0, eb1), (rb0, rb1)
    gss, sss, iss = (gs0, gs1), (ss0, ss1), (is0, is1)

    for t in range(NPT // WCH):
        r0 = s * NPT + t * WCH
        pltpu.sync_copy(Z.at[pl.ds(r0, WCH), pl.ds(96 + c * D, D)],
                        rb0.at[pl.ds(0, WCH), :])
        pltpu.sync_copy(d1e.at[pl.ds(r0, WCH), :],
                        rb0.at[pl.ds(WCH, WCH), :])

        def crow(j, _):
            rb0[j] = rb0[j] * rb0[WCH + j]
            return ()

        lax.fori_loop(0, WCH, crow, ())
        pltpu.sync_copy(rb0.at[pl.ds(0, WCH), :],
                        g4.at[c].at[pl.ds(r0, WCH), :])
    plsc.subcore_barrier()

    for (rnd, tbl, gout, zc) in ((4, g4, w2, 64), (5, w2, w3, 32),
                                 (6, w3, None, 0)):
        _edge_sweep(tbl, eidx, accum, ebs, rbs, gss, sss, iss, c, s, rb0)
        nt = NPT // WCH

        def wb_writes(t):
            A = rbs[t % 2]
            r0 = s * NPT + t * WCH
            if gout is not None:
                return [(A.at[pl.ds(0, WCH), :],
                         gout.at[c].at[pl.ds(r0, WCH), :])]
            return [(A.at[pl.ds(0, WCH), :],
                     o32.at[pl.ds(r0, WCH), pl.ds(c * D, D)])]

        for t in range(nt):
            A = rbs[t % 2]
            sem = iss[t % 2]
            r0 = s * NPT + t * WCH
            if t >= 2:
                for (sr, dr) in wb_writes(t - 2):
                    pltpu.make_async_copy(sr, dr, sem).wait()
            pltpu.sync_copy(accum.at[pl.ds(r0, WCH), :],
                            A.at[pl.ds(0, WCH), :])
            pltpu.sync_copy(Z.at[pl.ds(r0, WCH), pl.ds(zc + c * D, D)],
                            A.at[pl.ds(WCH, WCH), :])
            pltpu.sync_copy(d1e.at[pl.ds(r0, WCH), :],
                            A.at[pl.ds(2 * WCH, WCH), :])
            if gout is not None:
                def wrow(j, _):
                    d1 = A[2 * WCH + j]
                    A[j] = d1 * (d1 * A[j] + A[WCH + j])
                    return ()
            else:
                def wrow(j, _):
                    v = A[WCH + j] + A[j] * A[2 * WCH + j]
                    A[j] = jnp.where(v > 0, v, 0.25 * v)
                    return ()

            lax.fori_loop(0, WCH, wrow, ())
            for (sr, dr) in wb_writes(t):
                pltpu.async_copy(sr, dr, sem)
        for t in (nt - 2, nt - 1):
            for (sr, dr) in wb_writes(t):
                pltpu.make_async_copy(sr, dr, iss[t % 2]).wait()
        plsc.subcore_barrier()


_PROP_SCRATCH = dict(
    accum=pltpu.VMEM_SHARED((N, D), _f32),
    eb0=pltpu.VMEM((2, CH), jnp.int32),
    eb1=pltpu.VMEM((2, CH), jnp.int32),
    rb0=pltpu.VMEM((CH, D), _f32),
    rb1=pltpu.VMEM((CH, D), _f32),
    gs0=pltpu.SemaphoreType.DMA,
    gs1=pltpu.SemaphoreType.DMA,
    ss0=pltpu.SemaphoreType.DMA,
    ss1=pltpu.SemaphoreType.DMA,
    is0=pltpu.SemaphoreType.DMA,
    is1=pltpu.SemaphoreType.DMA,
)


def _run_l1(eidx, x32, d1e):
    gt = jax.ShapeDtypeStruct((NC, N, D), _f32)
    k = pl.kernel(
        _l1_body,
        out_type=(gt, gt, gt, jax.ShapeDtypeStruct((N, 128), _f32)),
        mesh=_sc_mesh(),
        scratch_types=dict(_PROP_SCRATCH),
        compiler_params=_SC_PARAMS,
        name="sc_prop_l1",
    )
    return k(eidx, x32, d1e)


def _run_l2(eidx, Z, d1e):
    gt = jax.ShapeDtypeStruct((NC, N, D), _f32)
    k = pl.kernel(
        _l2_body,
        out_type=(gt, gt, gt, jax.ShapeDtypeStruct((N, 32), _f32)),
        mesh=_sc_mesh(),
        scratch_types=dict(_PROP_SCRATCH),
        compiler_params=_SC_PARAMS,
        name="sc_prop_l2",
    )
    return k(eidx, Z, d1e)


BM = 2000


def _main_body(u_ref, w1t_ref, b1_ref, w2r_ref, b2p_ref, z_ref):
    y = jnp.dot(u_ref[...], w1t_ref[...],
                preferred_element_type=_f32) + b1_ref[...]
    h = jnp.where(y > 0, y, 0.25 * y)
    z_ref[...] = jnp.dot(h, w2r_ref[...],
                         preferred_element_type=_f32) + b2p_ref[...]


def _tc_main(u128, W1, b1, W2, b2):
    w1t = W1.T
    w2r = jnp.concatenate(
        [W2[:, 64 * j:64 * (j + 1)].T for j in range(4)], axis=1)
    b2p = jnp.concatenate([b2, jnp.zeros((96,), _f32)]).reshape(1, 128)
    return pl.pallas_call(
        _main_body,
        grid=(N // BM,),
        in_specs=[
            pl.BlockSpec((BM, 128), lambda i: (i, 0)),
            pl.BlockSpec((128, 64), lambda i: (0, 0)),
            pl.BlockSpec((1, 64), lambda i: (0, 0)),
            pl.BlockSpec((64, 128), lambda i: (0, 0)),
            pl.BlockSpec((1, 128), lambda i: (0, 0)),
        ],
        out_specs=pl.BlockSpec((BM, 128), lambda i: (i, 0)),
        out_shape=jax.ShapeDtypeStruct((N, 128), _f32),
    )(u128, w1t, b1.reshape(1, 64), w2r, b2p)


def kernel(category, noise, edge_index, W1, b1, W2, b2):
    eidx = edge_index.astype(jnp.int32)
    dst = eidx[1]
    ones16 = jnp.ones((DCH, D), _f32)
    x32 = jnp.concatenate([category, noise], axis=1)

    d1e = _degree(dst, ones16)

    _, _, _, u128 = _run_l1(eidx, x32, d1e)
    Z = _tc_main(u128, W1, b1, W2, b2)
    _, _, _, o32 = _run_l2(eidx, Z, d1e)
    return o32


# speedup vs baseline: 1.4379x; 1.0636x over previous
import functools

import jax
import jax.numpy as jnp
from jax import lax
from jax.experimental import pallas as pl
from jax.experimental.pallas import tpu as pltpu
from jax.experimental.pallas import tpu_sc as plsc

N = 100000
E = 1600000
NC = 2
NS = 16
NW = NC * NS
D = 16
CH = 800
NCHUNK = E // NS // CH
EPT = E // NS
WCH = 250
NPT = N // NS
DCH = 1000
DNCHUNK = EPT // DCH

_f32 = jnp.float32


def _sc_mesh():
    return plsc.VectorSubcoreMesh(
        core_axis_name="c", subcore_axis_name="s", num_cores=NC, num_subcores=NS
    )


_SC_PARAMS = pltpu.CompilerParams(
    use_tc_tiling_on_sc=False, needs_layout_passes=False)


def _rsqrt_nr(v):
    i = plsc.bitcast(v, jnp.int32)
    y = plsc.bitcast(jnp.int32(0x5F3759DF) - (i >> 1), _f32)
    for _ in range(4):
        y = y * (1.5 - 0.5 * v * y * y)
    return y


def _deg_kernel(dst, ones16, d1e, shared, didx0, didx1,
                sbufd, d1b, obuf, is0, is1):
    c = lax.axis_index("c")
    s = lax.axis_index("s")
    wid = c * NS + s
    pltpu.sync_copy(ones16, obuf)

    def zb(j, _):
        d1b[j] = jnp.zeros((16,), _f32)
        return ()

    lax.fori_loop(0, WCH, zb, ())
    for t in range(NPT // WCH):
        pltpu.sync_copy(d1b, shared.at[pl.ds(s * NPT + t * WCH, WCH), :])
    plsc.subcore_barrier()

    dbs = (didx0, didx1)
    iss = (is0, is1)

    def idx_start(i, p):
        base = s * EPT + i * DCH
        return pltpu.async_copy(dst.at[pl.ds(base, DCH)], dbs[p], iss[p])

    def idx_wait(i, p):
        base = s * EPT + i * DCH
        pltpu.make_async_copy(dst.at[pl.ds(base, DCH)], dbs[p], iss[p]).wait()

    def scat(p):
        pltpu.sync_copy(obuf, shared.at[dbs[p]], add=True)

    idx_start(0, 0)

    def pair(k, _):
        i0 = 2 * k
        idx_wait(i0, 0)
        idx_start(i0 + 1, 1)
        scat(0)
        idx_wait(i0 + 1, 1)
        idx_start(i0 + 2, 0)
        scat(1)
        return ()

    assert DNCHUNK % 2 == 0
    lax.fori_loop(0, DNCHUNK // 2 - 1, pair, ())
    i0 = DNCHUNK - 2
    idx_wait(i0, 0)
    idx_start(i0 + 1, 1)
    scat(0)
    idx_wait(i0 + 1, 1)
    scat(1)
    plsc.subcore_barrier()

    wch2 = N // NW // 25
    for t in range(25):
        r0 = wid * (N // NW) + t * wch2
        pltpu.sync_copy(shared.at[pl.ds(r0, wch2), :],
                        sbufd.at[pl.ds(0, wch2), :])

        def srow(j, _):
            v = sbufd[j]
            pos = v > 0
            safe = jnp.where(pos, v, 1.0)
            d1b[j] = jnp.where(pos, _rsqrt_nr(safe), 0.0)
            return ()

        lax.fori_loop(0, wch2, srow, ())
        pltpu.sync_copy(d1b.at[pl.ds(0, wch2), :], d1e.at[pl.ds(r0, wch2), :])


def _degree(dst, ones16):
    k = pl.kernel(
        _deg_kernel,
        out_type=jax.ShapeDtypeStruct((N, D), _f32),
        mesh=_sc_mesh(),
        scratch_types=dict(
            shared=pltpu.VMEM_SHARED((N, D), _f32),
            didx0=pltpu.VMEM((DCH,), jnp.int32),
            didx1=pltpu.VMEM((DCH,), jnp.int32),
            sbufd=pltpu.VMEM((WCH, D), _f32),
            d1b=pltpu.VMEM((WCH, D), _f32),
            obuf=pltpu.VMEM((DCH, D), _f32),
            is0=pltpu.SemaphoreType.DMA,
            is1=pltpu.SemaphoreType.DMA,
        ),
        compiler_params=_SC_PARAMS,
        name="sc_deg",
    )
    return k(dst, ones16)


def _edge_sweep(tbl, eidx, accum, ebs, rbs, gss, sss, iss, c, s, rb0):
    def zb(j, _):
        rb0[j] = jnp.zeros((16,), _f32)
        return ()

    lax.fori_loop(0, WCH, zb, ())
    for t in range(NPT // WCH):
        pltpu.sync_copy(rb0.at[pl.ds(0, WCH), :],
                        accum.at[pl.ds(s * NPT + t * WCH, WCH), :])
    plsc.subcore_barrier()

    def idx_start(i, p):
        base = s * EPT + i * CH
        return pltpu.async_copy(eidx.at[:, pl.ds(base, CH)], ebs[p], iss[p])

    def gather_start(p):
        return pltpu.async_copy(tbl.at[c].at[ebs[p].at[0]], rbs[p], gss[p])

    def gather_wait(p):
        pltpu.make_async_copy(tbl.at[c].at[ebs[p].at[0]], rbs[p], gss[p]).wait()

    def scat_start(p):
        return pltpu.async_copy(rbs[p], accum.at[ebs[p].at[1]], sss[p], add=True)

    def scat_wait(p):
        pltpu.make_async_copy(rbs[p], accum.at[ebs[p].at[1]], sss[p]).wait()

    idx_start(0, 0).wait()
    gather_start(0)
    idx_start(1, 1).wait()
    gather_wait(0)
    scat_start(0)
    gather_start(1)

    def chunk_body(i, p):
        scat_wait(p)
        idesc = idx_start(i, p)
        gather_wait(1 - p)
        scat_start(1 - p)
        idesc.wait()
        gather_start(p)
        return ()

    def pair(k, _):
        i0 = 2 + 2 * k
        chunk_body(i0, 0)
        chunk_body(i0 + 1, 1)
        return ()

    lax.fori_loop(0, (NCHUNK - 2) // 2, pair, ())
    if (NCHUNK - 2) % 2 == 1:
        chunk_body(NCHUNK - 1, 0)
        last = 0
    else:
        last = 1
    gather_wait(last)
    scat_start(last)
    scat_wait(1 - last)
    scat_wait(last)
    plsc.subcore_barrier()


def _l1_body(eidx, x32, d1e, g0, g1, g2, u128, accum,
             eb0, eb1, rb0, rb1, gs0, gs1, ss0, ss1, is0, is1):
    c = lax.axis_index("c")
    s = lax.axis_index("s")
    ebs, rbs = (eb0, eb1), (rb0, rb1)
    gss, sss, iss = (gs0, gs1), (ss0, ss1), (is0, is1)

    for t in range(NPT // WCH):
        r0 = s * NPT + t * WCH
        for half in range(2):
            pltpu.sync_copy(x32.at[pl.ds(r0, WCH), pl.ds(16 * half, 16)],
                            rb1.at[pl.ds(0, WCH), :])
            pltpu.sync_copy(rb1.at[pl.ds(0, WCH), :],
                            u128.at[pl.ds(r0, WCH), pl.ds(16 * half, 16)])
        pltpu.sync_copy(x32.at[pl.ds(r0, WCH), pl.ds(c * D, D)],
                        rb0.at[pl.ds(0, WCH), :])
        pltpu.sync_copy(d1e.at[pl.ds(r0, WCH), :],
                        rb0.at[pl.ds(WCH, WCH), :])

        def crow(j, _):
            for u in range(2):
                rb0[2 * j + u] = rb0[2 * j + u] * rb0[WCH + 2 * j + u]
            return ()

        lax.fori_loop(0, WCH // 2, crow, ())
        pltpu.sync_copy(rb0.at[pl.ds(0, WCH), :],
                        g0.at[c].at[pl.ds(r0, WCH), :])
    plsc.subcore_barrier()

    for (rnd, tbl, gout) in ((1, g0, g1), (2, g1, g2), (3, g2, None)):
        _edge_sweep(tbl, eidx, accum, ebs, rbs, gss, sss, iss, c, s, rb0)
        nt = NPT // WCH

        def wb_writes(t):
            A = rbs[t % 2]
            r0 = s * NPT + t * WCH
            outs = [(A.at[pl.ds(2 * WCH, WCH), :],
                     u128.at[pl.ds(r0, WCH), pl.ds(32 * rnd + c * D, D)])]
            if gout is not None:
                outs.append((A.at[pl.ds(0, WCH), :],
                             gout.at[c].at[pl.ds(r0, WCH), :]))
            return outs

        for t in range(nt):
            A = rbs[t % 2]
            sem = iss[t % 2]
            r0 = s * NPT + t * WCH
            if t >= 2:
                for (sr, dr) in wb_writes(t - 2):
                    pltpu.make_async_copy(sr, dr, sem).wait()
            pltpu.sync_copy(accum.at[pl.ds(r0, WCH), :],
                            A.at[pl.ds(0, WCH), :])
            pltpu.sync_copy(d1e.at[pl.ds(r0, WCH), :],
                            A.at[pl.ds(WCH, WCH), :])

            def wrow(j, _):
                for u in range(2):
                    a = A[2 * j + u] * A[WCH + 2 * j + u]
                    A[2 * WCH + 2 * j + u] = a
                    if gout is not None:
                        A[2 * j + u] = a * A[WCH + 2 * j + u]
                return ()

            lax.fori_loop(0, WCH // 2, wrow, ())
            for (sr, dr) in wb_writes(t):
                pltpu.async_copy(sr, dr, sem)
        for t in (nt - 2, nt - 1):
            for (sr, dr) in wb_writes(t):
                pltpu.make_async_copy(sr, dr, iss[t % 2]).wait()
        plsc.subcore_barrier()


def _l2_body(eidx, Z, d1e, g4, w2, w3, o32, accum,
             eb0, eb1, rb0, rb1, gs0, gs1, ss0, ss1, is0, is1):
    c = lax.axis_index("c")
    s = lax.axis_index("s")
    ebs, rbs = (eb0, eb1), (rb0, rb1)
    gss, sss, iss = (gs0, gs1), (ss0, ss1), (is0, is1)

    for t in range(NPT // WCH):
        r0 = s * NPT + t * WCH
        pltpu.sync_copy(Z.at[pl.ds(r0, WCH), pl.ds(96 + c * D, D)],
                        rb0.at[pl.ds(0, WCH), :])
        pltpu.sync_copy(d1e.at[pl.ds(r0, WCH), :],
                        rb0.at[pl.ds(WCH, WCH), :])

        def crow(j, _):
            for u in range(2):
                rb0[2 * j + u] = rb0[2 * j + u] * rb0[WCH + 2 * j + u]
            return ()

        lax.fori_loop(0, WCH // 2, crow, ())
        pltpu.sync_copy(rb0.at[pl.ds(0, WCH), :],
                        g4.at[c].at[pl.ds(r0, WCH), :])
    plsc.subcore_barrier()

    for (rnd, tbl, gout, zc) in ((4, g4, w2, 64), (5, w2, w3, 32),
                                 (6, w3, None, 0)):
        _edge_sweep(tbl, eidx, accum, ebs, rbs, gss, sss, iss, c, s, rb0)
        nt = NPT // WCH

        def wb_writes(t):
            A = rbs[t % 2]
            r0 = s * NPT + t * WCH
            if gout is not None:
                return [(A.at[pl.ds(0, WCH), :],
                         gout.at[c].at[pl.ds(r0, WCH), :])]
            return [(A.at[pl.ds(0, WCH), :],
                     o32.at[pl.ds(r0, WCH), pl.ds(c * D, D)])]

        for t in range(nt):
            A = rbs[t % 2]
            sem = iss[t % 2]
            r0 = s * NPT + t * WCH
            if t >= 2:
                for (sr, dr) in wb_writes(t - 2):
                    pltpu.make_async_copy(sr, dr, sem).wait()
            pltpu.sync_copy(accum.at[pl.ds(r0, WCH), :],
                            A.at[pl.ds(0, WCH), :])
            pltpu.sync_copy(Z.at[pl.ds(r0, WCH), pl.ds(zc + c * D, D)],
                            A.at[pl.ds(WCH, WCH), :])
            pltpu.sync_copy(d1e.at[pl.ds(r0, WCH), :],
                            A.at[pl.ds(2 * WCH, WCH), :])
            if gout is not None:
                def wrow(j, _):
                    for u in range(2):
                        d1 = A[2 * WCH + 2 * j + u]
                        A[2 * j + u] = d1 * (d1 * A[2 * j + u]
                                             + A[WCH + 2 * j + u])
                    return ()
            else:
                def wrow(j, _):
                    for u in range(2):
                        v = (A[WCH + 2 * j + u]
                             + A[2 * j + u] * A[2 * WCH + 2 * j + u])
                        A[2 * j + u] = jnp.where(v > 0, v, 0.25 * v)
                    return ()

            lax.fori_loop(0, WCH // 2, wrow, ())
            for (sr, dr) in wb_writes(t):
                pltpu.async_copy(sr, dr, sem)
        for t in (nt - 2, nt - 1):
            for (sr, dr) in wb_writes(t):
                pltpu.make_async_copy(sr, dr, iss[t % 2]).wait()
        plsc.subcore_barrier()


_PROP_SCRATCH = dict(
    accum=pltpu.VMEM_SHARED((N, D), _f32),
    eb0=pltpu.VMEM((2, CH), jnp.int32),
    eb1=pltpu.VMEM((2, CH), jnp.int32),
    rb0=pltpu.VMEM((CH, D), _f32),
    rb1=pltpu.VMEM((CH, D), _f32),
    gs0=pltpu.SemaphoreType.DMA,
    gs1=pltpu.SemaphoreType.DMA,
    ss0=pltpu.SemaphoreType.DMA,
    ss1=pltpu.SemaphoreType.DMA,
    is0=pltpu.SemaphoreType.DMA,
    is1=pltpu.SemaphoreType.DMA,
)


def _run_l1(eidx, x32, d1e):
    gt = jax.ShapeDtypeStruct((NC, N, D), _f32)
    k = pl.kernel(
        _l1_body,
        out_type=(gt, gt, gt, jax.ShapeDtypeStruct((N, 128), _f32)),
        mesh=_sc_mesh(),
        scratch_types=dict(_PROP_SCRATCH),
        compiler_params=_SC_PARAMS,
        name="sc_prop_l1",
    )
    return k(eidx, x32, d1e)


def _run_l2(eidx, Z, d1e):
    gt = jax.ShapeDtypeStruct((NC, N, D), _f32)
    k = pl.kernel(
        _l2_body,
        out_type=(gt, gt, gt, jax.ShapeDtypeStruct((N, 32), _f32)),
        mesh=_sc_mesh(),
        scratch_types=dict(_PROP_SCRATCH),
        compiler_params=_SC_PARAMS,
        name="sc_prop_l2",
    )
    return k(eidx, Z, d1e)


BM = 2000


def _main_body(u_ref, w1t_ref, b1_ref, w2r_ref, b2p_ref, z_ref):
    y = jnp.dot(u_ref[...], w1t_ref[...],
                preferred_element_type=_f32) + b1_ref[...]
    h = jnp.where(y > 0, y, 0.25 * y)
    z_ref[...] = jnp.dot(h, w2r_ref[...],
                         preferred_element_type=_f32) + b2p_ref[...]


def _tc_main(u128, W1, b1, W2, b2):
    w1t = W1.T
    w2r = jnp.concatenate(
        [W2[:, 64 * j:64 * (j + 1)].T for j in range(4)], axis=1)
    b2p = jnp.concatenate([b2, jnp.zeros((96,), _f32)]).reshape(1, 128)
    return pl.pallas_call(
        _main_body,
        grid=(N // BM,),
        in_specs=[
            pl.BlockSpec((BM, 128), lambda i: (i, 0)),
            pl.BlockSpec((128, 64), lambda i: (0, 0)),
            pl.BlockSpec((1, 64), lambda i: (0, 0)),
            pl.BlockSpec((64, 128), lambda i: (0, 0)),
            pl.BlockSpec((1, 128), lambda i: (0, 0)),
        ],
        out_specs=pl.BlockSpec((BM, 128), lambda i: (i, 0)),
        out_shape=jax.ShapeDtypeStruct((N, 128), _f32),
    )(u128, w1t, b1.reshape(1, 64), w2r, b2p)


def kernel(category, noise, edge_index, W1, b1, W2, b2):
    eidx = edge_index.astype(jnp.int32)
    dst = eidx[1]
    ones16 = jnp.ones((DCH, D), _f32)
    x32 = jnp.concatenate([category, noise], axis=1)

    d1e = _degree(dst, ones16)

    _, _, _, u128 = _run_l1(eidx, x32, d1e)
    Z = _tc_main(u128, W1, b1, W2, b2)
    _, _, _, o32 = _run_l2(eidx, Z, d1e)
    return o32


# speedup vs baseline: 1.4845x; 1.0324x over previous
import functools

import jax
import jax.numpy as jnp
from jax import lax
from jax.experimental import pallas as pl
from jax.experimental.pallas import tpu as pltpu
from jax.experimental.pallas import tpu_sc as plsc

N = 100000
E = 1600000
NC = 2
NS = 16
NW = NC * NS
D = 16
CH = 800
NCHUNK = E // NS // CH
EPT = E // NS
WCH = 250
NPT = N // NS
DCH = 1000
DNCHUNK = EPT // DCH

_f32 = jnp.float32


def _sc_mesh():
    return plsc.VectorSubcoreMesh(
        core_axis_name="c", subcore_axis_name="s", num_cores=NC, num_subcores=NS
    )


_SC_PARAMS = pltpu.CompilerParams(
    use_tc_tiling_on_sc=False, needs_layout_passes=False)


def _rsqrt_nr(v):
    i = plsc.bitcast(v, jnp.int32)
    y = plsc.bitcast(jnp.int32(0x5F3759DF) - (i >> 1), _f32)
    for _ in range(4):
        y = y * (1.5 - 0.5 * v * y * y)
    return y


def _deg_kernel(dst, ones16, d1e, shared, didx0, didx1,
                sbufd, d1b, obuf, is0, is1):
    c = lax.axis_index("c")
    s = lax.axis_index("s")
    wid = c * NS + s
    pltpu.sync_copy(ones16, obuf)

    def zb(j, _):
        d1b[j] = jnp.zeros((16,), _f32)
        return ()

    lax.fori_loop(0, WCH, zb, ())
    for t in range(NPT // WCH):
        pltpu.sync_copy(d1b, shared.at[pl.ds(s * NPT + t * WCH, WCH), :])
    plsc.subcore_barrier()

    dbs = (didx0, didx1)
    iss = (is0, is1)

    def idx_start(i, p):
        base = s * EPT + i * DCH
        return pltpu.async_copy(dst.at[pl.ds(base, DCH)], dbs[p], iss[p])

    def idx_wait(i, p):
        base = s * EPT + i * DCH
        pltpu.make_async_copy(dst.at[pl.ds(base, DCH)], dbs[p], iss[p]).wait()

    def scat(p):
        pltpu.sync_copy(obuf, shared.at[dbs[p]], add=True)

    idx_start(0, 0)

    def pair(k, _):
        i0 = 2 * k
        idx_wait(i0, 0)
        idx_start(i0 + 1, 1)
        scat(0)
        idx_wait(i0 + 1, 1)
        idx_start(i0 + 2, 0)
        scat(1)
        return ()

    assert DNCHUNK % 2 == 0
    lax.fori_loop(0, DNCHUNK // 2 - 1, pair, ())
    i0 = DNCHUNK - 2
    idx_wait(i0, 0)
    idx_start(i0 + 1, 1)
    scat(0)
    idx_wait(i0 + 1, 1)
    scat(1)
    plsc.subcore_barrier()

    wch2 = N // NW // 25
    for t in range(25):
        r0 = wid * (N // NW) + t * wch2
        pltpu.sync_copy(shared.at[pl.ds(r0, wch2), :],
                        sbufd.at[pl.ds(0, wch2), :])

        def srow(j, _):
            v = sbufd[j]
            pos = v > 0
            safe = jnp.where(pos, v, 1.0)
            d1b[j] = jnp.where(pos, _rsqrt_nr(safe), 0.0)
            return ()

        lax.fori_loop(0, wch2, srow, ())
        pltpu.sync_copy(d1b.at[pl.ds(0, wch2), :], d1e.at[pl.ds(r0, wch2), :])


def _degree(dst, ones16):
    k = pl.kernel(
        _deg_kernel,
        out_type=jax.ShapeDtypeStruct((N, D), _f32),
        mesh=_sc_mesh(),
        scratch_types=dict(
            shared=pltpu.VMEM_SHARED((N, D), _f32),
            didx0=pltpu.VMEM((DCH,), jnp.int32),
            didx1=pltpu.VMEM((DCH,), jnp.int32),
            sbufd=pltpu.VMEM((WCH, D), _f32),
            d1b=pltpu.VMEM((WCH, D), _f32),
            obuf=pltpu.VMEM((DCH, D), _f32),
            is0=pltpu.SemaphoreType.DMA,
            is1=pltpu.SemaphoreType.DMA,
        ),
        compiler_params=_SC_PARAMS,
        name="sc_deg",
    )
    return k(dst, ones16)


def _edge_sweep(tbl, eidx, accum, ebs, rbs, gss, sss, iss, c, s, rb0):
    def zb(j, _):
        rb0[j] = jnp.zeros((16,), _f32)
        return ()

    lax.fori_loop(0, WCH, zb, ())
    for t in range(NPT // WCH):
        pltpu.sync_copy(rb0.at[pl.ds(0, WCH), :],
                        accum.at[pl.ds(s * NPT + t * WCH, WCH), :])
    plsc.subcore_barrier()

    def idx_start(i, p):
        base = s * EPT + i * CH
        return pltpu.async_copy(eidx.at[:, pl.ds(base, CH)], ebs[p], iss[p])

    def gather_start(p):
        return pltpu.async_copy(tbl.at[c].at[ebs[p].at[0]], rbs[p], gss[p])

    def gather_wait(p):
        pltpu.make_async_copy(tbl.at[c].at[ebs[p].at[0]], rbs[p], gss[p]).wait()

    def scat_start(p):
        return pltpu.async_copy(rbs[p], accum.at[ebs[p].at[1]], sss[p], add=True)

    def scat_wait(p):
        pltpu.make_async_copy(rbs[p], accum.at[ebs[p].at[1]], sss[p]).wait()

    idx_start(0, 0).wait()
    gather_start(0)
    idx_start(1, 1).wait()
    gather_wait(0)
    scat_start(0)
    gather_start(1)

    def chunk_body(i, p):
        scat_wait(p)
        idesc = idx_start(i, p)
        gather_wait(1 - p)
        scat_start(1 - p)
        idesc.wait()
        gather_start(p)
        return ()

    def pair(k, _):
        i0 = 2 + 2 * k
        chunk_body(i0, 0)
        chunk_body(i0 + 1, 1)
        return ()

    lax.fori_loop(0, (NCHUNK - 2) // 2, pair, ())
    if (NCHUNK - 2) % 2 == 1:
        chunk_body(NCHUNK - 1, 0)
        last = 0
    else:
        last = 1
    gather_wait(last)
    scat_start(last)
    scat_wait(1 - last)
    scat_wait(last)
    plsc.subcore_barrier()


def _l1_body(eidx, x32, d1e, g0, g1, g2, u128, accum,
             eb0, eb1, rb0, rb1, gs0, gs1, ss0, ss1, is0, is1):
    c = lax.axis_index("c")
    s = lax.axis_index("s")
    ebs, rbs = (eb0, eb1), (rb0, rb1)
    gss, sss, iss = (gs0, gs1), (ss0, ss1), (is0, is1)

    for t in range(NPT // WCH):
        r0 = s * NPT + t * WCH
        for half in range(2):
            pltpu.sync_copy(x32.at[pl.ds(r0, WCH), pl.ds(16 * half, 16)],
                            rb1.at[pl.ds(0, WCH), :])
            pltpu.sync_copy(rb1.at[pl.ds(0, WCH), :],
                            u128.at[pl.ds(r0, WCH), pl.ds(16 * half, 16)])
        pltpu.sync_copy(x32.at[pl.ds(r0, WCH), pl.ds(c * D, D)],
                        rb0.at[pl.ds(0, WCH), :])
        pltpu.sync_copy(d1e.at[pl.ds(r0, WCH), :],
                        rb0.at[pl.ds(WCH, WCH), :])

        def crow(j, _):
            for u in range(5):
                rb0[5 * j + u] = rb0[5 * j + u] * rb0[WCH + 5 * j + u]
            return ()

        lax.fori_loop(0, WCH // 5, crow, ())
        pltpu.sync_copy(rb0.at[pl.ds(0, WCH), :],
                        g0.at[c].at[pl.ds(r0, WCH), :])
    plsc.subcore_barrier()

    for (rnd, tbl, gout) in ((1, g0, g1), (2, g1, g2), (3, g2, None)):
        _edge_sweep(tbl, eidx, accum, ebs, rbs, gss, sss, iss, c, s, rb0)
        nt = NPT // WCH

        def wb_writes(t):
            A = rbs[t % 2]
            r0 = s * NPT + t * WCH
            outs = [(A.at[pl.ds(2 * WCH, WCH), :],
                     u128.at[pl.ds(r0, WCH), pl.ds(32 * rnd + c * D, D)])]
            if gout is not None:
                outs.append((A.at[pl.ds(0, WCH), :],
                             gout.at[c].at[pl.ds(r0, WCH), :]))
            return outs

        for t in range(nt):
            A = rbs[t % 2]
            sem = iss[t % 2]
            r0 = s * NPT + t * WCH
            if t >= 2:
                for (sr, dr) in wb_writes(t - 2):
                    pltpu.make_async_copy(sr, dr, sem).wait()
            pltpu.sync_copy(accum.at[pl.ds(r0, WCH), :],
                            A.at[pl.ds(0, WCH), :])
            pltpu.sync_copy(d1e.at[pl.ds(r0, WCH), :],
                            A.at[pl.ds(WCH, WCH), :])

            def wrow(j, _):
                for u in range(5):
                    a = A[5 * j + u] * A[WCH + 5 * j + u]
                    A[2 * WCH + 5 * j + u] = a
                    if gout is not None:
                        A[5 * j + u] = a * A[WCH + 5 * j + u]
                return ()

            lax.fori_loop(0, WCH // 5, wrow, ())
            for (sr, dr) in wb_writes(t):
                pltpu.async_copy(sr, dr, sem)
        for t in (nt - 2, nt - 1):
            for (sr, dr) in wb_writes(t):
                pltpu.make_async_copy(sr, dr, iss[t % 2]).wait()
        plsc.subcore_barrier()


def _l2_body(eidx, Z, d1e, g4, w2, w3, o32, accum,
             eb0, eb1, rb0, rb1, gs0, gs1, ss0, ss1, is0, is1):
    c = lax.axis_index("c")
    s = lax.axis_index("s")
    ebs, rbs = (eb0, eb1), (rb0, rb1)
    gss, sss, iss = (gs0, gs1), (ss0, ss1), (is0, is1)

    for t in range(NPT // WCH):
        r0 = s * NPT + t * WCH
        pltpu.sync_copy(Z.at[pl.ds(r0, WCH), pl.ds(96 + c * D, D)],
                        rb0.at[pl.ds(0, WCH), :])
        pltpu.sync_copy(d1e.at[pl.ds(r0, WCH), :],
                        rb0.at[pl.ds(WCH, WCH), :])

        def crow(j, _):
            for u in range(5):
                rb0[5 * j + u] = rb0[5 * j + u] * rb0[WCH + 5 * j + u]
            return ()

        lax.fori_loop(0, WCH // 5, crow, ())
        pltpu.sync_copy(rb0.at[pl.ds(0, WCH), :],
                        g4.at[c].at[pl.ds(r0, WCH), :])
    plsc.subcore_barrier()

    for (rnd, tbl, gout, zc) in ((4, g4, w2, 64), (5, w2, w3, 32),
                                 (6, w3, None, 0)):
        _edge_sweep(tbl, eidx, accum, ebs, rbs, gss, sss, iss, c, s, rb0)
        nt = NPT // WCH

        def wb_writes(t):
            A = rbs[t % 2]
            r0 = s * NPT + t * WCH
            if gout is not None:
                return [(A.at[pl.ds(0, WCH), :],
                         gout.at[c].at[pl.ds(r0, WCH), :])]
            return [(A.at[pl.ds(0, WCH), :],
                     o32.at[pl.ds(r0, WCH), pl.ds(c * D, D)])]

        for t in range(nt):
            A = rbs[t % 2]
            sem = iss[t % 2]
            r0 = s * NPT + t * WCH
            if t >= 2:
                for (sr, dr) in wb_writes(t - 2):
                    pltpu.make_async_copy(sr, dr, sem).wait()
            pltpu.sync_copy(accum.at[pl.ds(r0, WCH), :],
                            A.at[pl.ds(0, WCH), :])
            pltpu.sync_copy(Z.at[pl.ds(r0, WCH), pl.ds(zc + c * D, D)],
                            A.at[pl.ds(WCH, WCH), :])
            pltpu.sync_copy(d1e.at[pl.ds(r0, WCH), :],
                            A.at[pl.ds(2 * WCH, WCH), :])
            if gout is not None:
                def wrow(j, _):
                    for u in range(5):
                        d1 = A[2 * WCH + 5 * j + u]
                        A[5 * j + u] = d1 * (d1 * A[5 * j + u]
                                             + A[WCH + 5 * j + u])
                    return ()
            else:
                def wrow(j, _):
                    for u in range(5):
                        v = (A[WCH + 5 * j + u]
                             + A[5 * j + u] * A[2 * WCH + 5 * j + u])
                        A[5 * j + u] = jnp.where(v > 0, v, 0.25 * v)
                    return ()

            lax.fori_loop(0, WCH // 5, wrow, ())
            for (sr, dr) in wb_writes(t):
                pltpu.async_copy(sr, dr, sem)
        for t in (nt - 2, nt - 1):
            for (sr, dr) in wb_writes(t):
                pltpu.make_async_copy(sr, dr, iss[t % 2]).wait()
        plsc.subcore_barrier()


_PROP_SCRATCH = dict(
    accum=pltpu.VMEM_SHARED((N, D), _f32),
    eb0=pltpu.VMEM((2, CH), jnp.int32),
    eb1=pltpu.VMEM((2, CH), jnp.int32),
    rb0=pltpu.VMEM((CH, D), _f32),
    rb1=pltpu.VMEM((CH, D), _f32),
    gs0=pltpu.SemaphoreType.DMA,
    gs1=pltpu.SemaphoreType.DMA,
    ss0=pltpu.SemaphoreType.DMA,
    ss1=pltpu.SemaphoreType.DMA,
    is0=pltpu.SemaphoreType.DMA,
    is1=pltpu.SemaphoreType.DMA,
)


def _run_l1(eidx, x32, d1e):
    gt = jax.ShapeDtypeStruct((NC, N, D), _f32)
    k = pl.kernel(
        _l1_body,
        out_type=(gt, gt, gt, jax.ShapeDtypeStruct((N, 128), _f32)),
        mesh=_sc_mesh(),
        scratch_types=dict(_PROP_SCRATCH),
        compiler_params=_SC_PARAMS,
        name="sc_prop_l1",
    )
    return k(eidx, x32, d1e)


def _run_l2(eidx, Z, d1e):
    gt = jax.ShapeDtypeStruct((NC, N, D), _f32)
    k = pl.kernel(
        _l2_body,
        out_type=(gt, gt, gt, jax.ShapeDtypeStruct((N, 32), _f32)),
        mesh=_sc_mesh(),
        scratch_types=dict(_PROP_SCRATCH),
        compiler_params=_SC_PARAMS,
        name="sc_prop_l2",
    )
    return k(eidx, Z, d1e)


BM = 2000


def _main_body(u_ref, w1t_ref, b1_ref, w2r_ref, b2p_ref, z_ref):
    y = jnp.dot(u_ref[...], w1t_ref[...],
                preferred_element_type=_f32) + b1_ref[...]
    h = jnp.where(y > 0, y, 0.25 * y)
    z_ref[...] = jnp.dot(h, w2r_ref[...],
                         preferred_element_type=_f32) + b2p_ref[...]


def _tc_main(u128, W1, b1, W2, b2):
    w1t = W1.T
    w2r = jnp.concatenate(
        [W2[:, 64 * j:64 * (j + 1)].T for j in range(4)], axis=1)
    b2p = jnp.concatenate([b2, jnp.zeros((96,), _f32)]).reshape(1, 128)
    return pl.pallas_call(
        _main_body,
        grid=(N // BM,),
        in_specs=[
            pl.BlockSpec((BM, 128), lambda i: (i, 0)),
            pl.BlockSpec((128, 64), lambda i: (0, 0)),
            pl.BlockSpec((1, 64), lambda i: (0, 0)),
            pl.BlockSpec((64, 128), lambda i: (0, 0)),
            pl.BlockSpec((1, 128), lambda i: (0, 0)),
        ],
        out_specs=pl.BlockSpec((BM, 128), lambda i: (i, 0)),
        out_shape=jax.ShapeDtypeStruct((N, 128), _f32),
    )(u128, w1t, b1.reshape(1, 64), w2r, b2p)


def kernel(category, noise, edge_index, W1, b1, W2, b2):
    eidx = edge_index.astype(jnp.int32)
    dst = eidx[1]
    ones16 = jnp.ones((DCH, D), _f32)
    x32 = jnp.concatenate([category, noise], axis=1)

    d1e = _degree(dst, ones16)

    _, _, _, u128 = _run_l1(eidx, x32, d1e)
    Z = _tc_main(u128, W1, b1, W2, b2)
    _, _, _, o32 = _run_l2(eidx, Z, d1e)
    return o32
